# baseline (speedup 1.0000x reference)
import functools

import jax
import jax.numpy as jnp
import numpy as np
from jax.experimental import pallas as pl

NP0, NP1 = 20, 20
NPREF = NP0 * NP1
NDIM = 3
B, S = 4, 4096
END_RATE = 0.05
INHIB = 0.1
NCH1, NCH2 = 5, 10

_CHUNK = 1024

_AR = np.arange(NP0)
_BASIS = ((((_AR[None, :] - _AR[:, None]) % NP0)[None, :, :]
           == _AR[:, None, None]).astype(np.float32))


def _circulant(k3):
    k20 = jnp.roll(jnp.pad(k3, ((0, NP0 - 3), (0, NP1 - 3))), (-1, -1), (0, 1))
    cx = jnp.einsum('uv,vcd->ucd', k20, _BASIS)
    return jnp.einsum('uab,ucd->acbd', _BASIS, cx).reshape(NPREF, NPREF)


def _up_bf16(a_bf16):
    bits = jax.lax.bitcast_convert_type(a_bf16, jnp.uint16).astype(jnp.uint32)
    return jax.lax.bitcast_convert_type(bits << 16, jnp.float32)


def _round_bf16_f32(a):
    return _up_bf16(a.astype(jnp.bfloat16))


def _bf16_limbs(a, n):
    limbs = []
    rem = a
    for _ in range(n):
        li = rem.astype(jnp.bfloat16)
        limbs.append(li)
        rem = rem - _up_bf16(li)
    return limbs


def _dot1(a_bf16, b_bf16):
    return jax.lax.dot_general(a_bf16, b_bf16, (((1,), (0,)), ((), ())),
                               preferred_element_type=jnp.float32)


def _net_kernel(x_ref, w1_ref, b1_ref, g1_ref, be1_ref, w2_ref, b2_ref,
                g2_ref, be2_ref, a1_ref, cb1_ref, m0_ref, m1_ref, m2_ref,
                c0_ref, wt_ref, xm2_ref):
    x = x_ref[0]
    xr = x.astype(jnp.bfloat16).astype(jnp.float32)
    w1 = w1_ref[...]
    h = (w1[:, 0:1] * xr[0:1, :] + w1[:, 1:2] * xr[1:2, :]
         + w1[:, 2:3] * xr[2:3, :]) + b1_ref[...]
    h = jnp.maximum(h, 0.0)
    mu = jnp.mean(h, axis=0, keepdims=True)
    var = jnp.mean((h - mu) ** 2, axis=0, keepdims=True)
    h = (h - mu) * jax.lax.rsqrt(var + 1e-5) * g1_ref[...] + be1_ref[...]

    h = _dot1(w2_ref[...], h.astype(jnp.bfloat16)) + b2_ref[...]
    h = jnp.maximum(h, 0.0)
    mu = jnp.mean(h, axis=0, keepdims=True)
    var = jnp.mean((h - mu) ** 2, axis=0, keepdims=True)
    h = (h - mu) * jax.lax.rsqrt(var + 1e-5) * g2_ref[...] + be2_ref[...]

    h3 = _dot1(a1_ref[...], h.astype(jnp.bfloat16)) + cb1_ref[...]
    bh3 = h3.astype(jnp.bfloat16)
    y = _dot1(m0_ref[...], bh3) + _dot1(m1_ref[...], bh3) \
        + _dot1(m2_ref[...], bh3)
    y = y * (1.0 / NCH2) + c0_ref[0, 0]
    wt_ref[0] = jax.nn.sigmoid(y)

    xm = (x[0:1, :] + x[1:2, :] + x[2:3, :]) * (1.0 / 3.0)
    xm2_ref[0] = xm * xm


def _select_kernel(wt_ref, xm2_ref, out_ref):
    iota = jax.lax.broadcasted_iota(jnp.int32, (NPREF, 1), 0)

    def body(state):
        t, resid, selmask, done = state
        new_resid, new_sel, new_done = [], [], []
        for b in range(B):
            rb = resid[b]
            rr = rb * rb * xm2_ref[b]
            wb = wt_ref[b]
            act2 = jnp.sum(wb * wb * rr, axis=1, keepdims=True)
            act2 = jnp.where(selmask[b] > 0.5, 0.0, act2)
            mx = jnp.max(act2)
            p = jnp.min(jnp.where(act2 == mx, iota, NPREF))
            onehot = (iota == p).astype(jnp.float32)
            already = jnp.max(onehot * selmask[b])
            active = 1.0 - done[b]
            wp = wt_ref[b, pl.ds(p, 1), :]
            gate = active * (1.0 - already)
            rb2 = jnp.maximum(rb - wp * gate, 0.0)
            new_resid.append(rb2)
            new_sel.append(jnp.maximum(selmask[b], onehot * active))
            new_done.append(jnp.maximum(
                done[b],
                jnp.where(jnp.mean(rb2) < END_RATE, 1.0, 0.0)))
        return (t + 1, tuple(new_resid), tuple(new_sel), tuple(new_done))

    def cond(state):
        t, _, _, done = state
        n_done = functools.reduce(jnp.add, done)
        return jnp.logical_and(t < NPREF, n_done < B - 0.5)

    state0 = (
        jnp.int32(0),
        tuple(jnp.ones((1, S), jnp.float32) for _ in range(B)),
        tuple(jnp.zeros((NPREF, 1), jnp.float32) for _ in range(B)),
        tuple(jnp.zeros((), jnp.float32) for _ in range(B)),
    )
    _, _, selmask, _ = jax.lax.while_loop(cond, body, state0)
    for b in range(B):
        scale = jnp.where(selmask[b] > 0.5, 1.0, INHIB)
        out_ref[b] = wt_ref[b] * scale


def kernel(x0, W1, b1, g1, be1, W2, b2, g2, be2, cw1, cb1, cw2, cb2):
    x0_t = jnp.transpose(x0, (0, 2, 1))
    col = lambda v: v.reshape(NPREF, 1)

    w1r = _round_bf16_f32(W1)
    w2r = W2.astype(jnp.bfloat16)
    a1 = jnp.concatenate([_circulant(cw1[ch, 0]) for ch in range(NCH1)],
                         axis=0).astype(jnp.bfloat16)
    cb1col = jnp.repeat(cb1, NPREF).reshape(NCH1 * NPREF, 1)
    c2f = _round_bf16_f32(cw2).sum(axis=0)
    a2 = jnp.concatenate([_circulant(c2f[i]) for i in range(NCH1)],
                         axis=1)
    m0, m1, m2 = _bf16_limbs(a2, 3)
    c0 = (cb2.sum() * (1.0 / NCH2)).reshape(1, 1)

    n_chunks = S // _CHUNK
    grid = (B * n_chunks,)
    full = lambda shape: pl.BlockSpec(shape, lambda i: (0,) * len(shape))
    wt, xm2 = pl.pallas_call(
        _net_kernel,
        grid=grid,
        in_specs=[
            pl.BlockSpec((1, NDIM, _CHUNK),
                         lambda i: (i // n_chunks, 0, i % n_chunks)),
            full((NPREF, NDIM)), full((NPREF, 1)), full((NPREF, 1)),
            full((NPREF, 1)), full((NPREF, NPREF)), full((NPREF, 1)),
            full((NPREF, 1)), full((NPREF, 1)),
            full((NCH1 * NPREF, NPREF)), full((NCH1 * NPREF, 1)),
            full((NPREF, NCH1 * NPREF)), full((NPREF, NCH1 * NPREF)),
            full((NPREF, NCH1 * NPREF)), full((1, 1)),
        ],
        out_specs=[
            pl.BlockSpec((1, NPREF, _CHUNK),
                         lambda i: (i // n_chunks, 0, i % n_chunks)),
            pl.BlockSpec((1, 1, _CHUNK),
                         lambda i: (i // n_chunks, 0, i % n_chunks)),
        ],
        out_shape=[
            jax.ShapeDtypeStruct((B, NPREF, S), jnp.float32),
            jax.ShapeDtypeStruct((B, 1, S), jnp.float32),
        ],
        name="pd_net",
    )(x0_t, w1r, col(b1), col(g1), col(be1), w2r, col(b2), col(g2), col(be2),
      a1, cb1col, m0, m1, m2, c0)

    out = pl.pallas_call(
        _select_kernel,
        out_shape=jax.ShapeDtypeStruct((B, NPREF, S), jnp.float32),
        input_output_aliases={0: 0},
        name="pd_select",
    )(wt, xm2)
    return out
